# baseline (speedup 1.0000x reference)
import functools

import jax
import jax.numpy as jnp
from jax.experimental import pallas as pl
from jax.experimental.pallas import tpu as pltpu

DIM = 64
VOCAB1 = 1000001
BLOCK = 16384
NBLK = -(-VOCAB1 // BLOCK)


def _lse_kernel(batch_ref, u_ref, v_ref, out_ref, acc_ref):
    i = pl.program_id(0)

    @pl.when(i == 0)
    def _init():
        acc_ref[0] = -jnp.inf
        acc_ref[1] = 0.0
        acc_ref[2] = 0.0

    u = u_ref[...].reshape(1, DIM)
    v = v_ref[...]
    z = jax.lax.dot_general(
        u, v, (((1,), (0,)), ((), ())), preferred_element_type=jnp.float32
    )

    col = i * BLOCK + jax.lax.broadcasted_iota(jnp.int32, (1, BLOCK), 1)
    valid = col < VOCAB1
    z = jnp.where(valid, z, -jnp.inf)

    y = batch_ref[1]
    acc_ref[2] += jnp.sum(jnp.where(col == y, z, 0.0))

    m_old = acc_ref[0]
    bmax = jnp.max(z)
    m_new = jnp.maximum(m_old, bmax)
    bsum = jnp.sum(jnp.exp(z - m_new))
    acc_ref[1] = acc_ref[1] * jnp.exp(m_old - m_new) + bsum
    acc_ref[0] = m_new

    @pl.when(i == NBLK - 1)
    def _finish():
        out_ref[0, 0] = (jnp.log(acc_ref[1]) + acc_ref[0]) - acc_ref[2]


@jax.jit
def _skipgram_loss(batch, u_table, v_table):
    grid_spec = pltpu.PrefetchScalarGridSpec(
        num_scalar_prefetch=1,
        grid=(NBLK,),
        in_specs=[
            pl.BlockSpec((1, 1, DIM), lambda i, b: (b[0], 0, 0)),
            pl.BlockSpec((DIM, BLOCK), lambda i, b: (0, i)),
        ],
        out_specs=pl.BlockSpec(memory_space=pltpu.SMEM),
        scratch_shapes=[pltpu.SMEM((4,), jnp.float32)],
    )
    out = pl.pallas_call(
        _lse_kernel,
        grid_spec=grid_spec,
        out_shape=jax.ShapeDtypeStruct((1, 1), jnp.float32),
    )(batch.astype(jnp.int32), u_table.reshape(-1, 1, DIM), v_table)
    return out[0, 0]


def kernel(batch, u_table, v_table):
    return _skipgram_loss(batch, u_table, v_table)
